# Initial kernel scaffold; baseline (speedup 1.0000x reference)
import jax
import jax.numpy as jnp
from jax.experimental import pallas as pl


def kernel(x, edge_index, W1, b1, W2, b2):
    raise NotImplementedError("write your pallas kernel here")


# speedup vs baseline: 5.0601x; 5.0601x over previous
import functools

import jax
import jax.numpy as jnp
from jax import lax
from jax.experimental import pallas as pl
from jax.experimental.pallas import tpu as pltpu
from jax.experimental.pallas import tpu_sc as plsc

N = 10000
E = 320000
D = 128
WPAD = 144

NC = 2
NS = 16
NW = NC * NS
EPT = E // NW
K = 80
NCH = EPT // K
RPT = N // NS
RB = 125

RBLK = 1000


def _roundtrip_scale(u):
    t2 = jnp.sum(u * u, axis=1, keepdims=True)
    t_raw = jnp.sqrt(t2)
    t = jnp.maximum(t_raw, 1e-7)
    e = jnp.exp(t)
    ei = 1.0 / e
    ch = 0.5 * (e + ei)
    sh = 0.5 * (e - ei)
    x0c = jnp.maximum(ch, 1.0 + 1e-7)
    dist = jnp.log(x0c + jnp.sqrt(x0c * x0c - 1.0))
    nrm = jnp.maximum(sh * (t_raw / t), 1e-7)
    return u * (dist * sh / (t * nrm))


def _tc1_body(x_ref, w_ref, b_ref, o_ref):
    u = _roundtrip_scale(x_ref[...])
    v = jnp.dot(u, w_ref[...], preferred_element_type=jnp.float32) + b_ref[...]
    r = v.shape[0]
    o_ref[...] = jnp.concatenate(
        [v, jnp.ones((r, 1), jnp.float32), jnp.zeros((r, WPAD - D - 1), jnp.float32)],
        axis=1,
    )


def _tc1(x, w, b):
    return pl.pallas_call(
        _tc1_body,
        grid=(N // RBLK,),
        in_specs=[
            pl.BlockSpec((RBLK, D), lambda i: (i, 0)),
            pl.BlockSpec((D, D), lambda i: (0, 0)),
            pl.BlockSpec((D,), lambda i: (0,)),
        ],
        out_specs=pl.BlockSpec((RBLK, WPAD), lambda i: (i, 0)),
        out_shape=jax.ShapeDtypeStruct((N, WPAD), jnp.float32),
    )(x, w, b)


def _tc2_body(p0_ref, p1_ref, w_ref, b_ref, tab_ref, deg_ref):
    s = p0_ref[...] + p1_ref[...]
    agg = s[:, :D]
    deg = jnp.maximum(s[:, D:D + 1], 1.0)
    u = jnp.maximum(agg / deg, 0.0)
    v = jnp.dot(_roundtrip_scale(u), w_ref[...], preferred_element_type=jnp.float32)
    tab_ref[...] = v + b_ref[...]
    deg_ref[...] = deg


def _tc2(p0, p1, w, b):
    return pl.pallas_call(
        _tc2_body,
        grid=(N // RBLK,),
        in_specs=[
            pl.BlockSpec((RBLK, WPAD), lambda i: (i, 0)),
            pl.BlockSpec((RBLK, WPAD), lambda i: (i, 0)),
            pl.BlockSpec((D, D), lambda i: (0, 0)),
            pl.BlockSpec((D,), lambda i: (0,)),
        ],
        out_specs=[
            pl.BlockSpec((RBLK, D), lambda i: (i, 0)),
            pl.BlockSpec((RBLK, 1), lambda i: (i, 0)),
        ],
        out_shape=[
            jax.ShapeDtypeStruct((N, D), jnp.float32),
            jax.ShapeDtypeStruct((N, 1), jnp.float32),
        ],
    )(p0, p1, w, b)


def _tc3_body(p0_ref, p1_ref, deg_ref, x0_ref, xr_ref):
    u = jnp.maximum((p0_ref[...] + p1_ref[...]) / deg_ref[...], 0.0)
    t_raw = jnp.sqrt(jnp.sum(u * u, axis=1, keepdims=True))
    t = jnp.maximum(t_raw, 1e-7)
    e = jnp.exp(t)
    ei = 1.0 / e
    x0_ref[...] = 0.5 * (e + ei)
    xr_ref[...] = 0.5 * (e - ei) * u / t


def _tc3(p0, p1, deg):
    return pl.pallas_call(
        _tc3_body,
        grid=(N // RBLK,),
        in_specs=[
            pl.BlockSpec((RBLK, D), lambda i: (i, 0)),
            pl.BlockSpec((RBLK, D), lambda i: (i, 0)),
            pl.BlockSpec((RBLK, 1), lambda i: (i, 0)),
        ],
        out_specs=[
            pl.BlockSpec((RBLK, 1), lambda i: (i, 0)),
            pl.BlockSpec((RBLK, D), lambda i: (i, 0)),
        ],
        out_shape=[
            jax.ShapeDtypeStruct((N, 1), jnp.float32),
            jax.ShapeDtypeStruct((N, D), jnp.float32),
        ],
    )(p0, p1, deg)


def _make_sc_agg(wd):
    mesh = plsc.VectorSubcoreMesh(core_axis_name="c", subcore_axis_name="s")

    @functools.partial(
        pl.kernel,
        out_type=jax.ShapeDtypeStruct((NC, N, wd), jnp.float32),
        mesh=mesh,
        scratch_types=[
            pltpu.VMEM((K,), jnp.int32),
            pltpu.VMEM((K,), jnp.int32),
            pltpu.VMEM((K, wd), jnp.float32),
            pltpu.VMEM((RB, wd), jnp.float32),
            pltpu.VMEM_SHARED((N, wd), jnp.float32),
            pltpu.SemaphoreType.DMA,
        ],
        compiler_params=pltpu.CompilerParams(use_tc_tiling_on_sc=False),
    )
    def sc_agg(tab_hbm, src_hbm, dst_hbm, out_hbm,
               sidx_v, didx_v, rows_v, zbuf_v, acc_sh, sem):
        c = lax.axis_index("c")
        s = lax.axis_index("s")
        wid = s * NC + c

        def zrow(r, carry):
            for j in range(wd // 16):
                zbuf_v[r, pl.ds(j * 16, 16)] = jnp.zeros((16,), jnp.float32)
            return carry

        lax.fori_loop(0, RB, zrow, 0)
        for cc in range(RPT // RB):
            pltpu.sync_copy(zbuf_v, acc_sh.at[pl.ds(s * RPT + cc * RB, RB), :])
        plsc.subcore_barrier()

        def body(i, carry):
            off = pl.multiple_of(wid * EPT + i * K, 8)
            pltpu.sync_copy(src_hbm.at[pl.ds(off, K)], sidx_v)
            pltpu.sync_copy(dst_hbm.at[pl.ds(off, K)], didx_v)
            pltpu.async_copy(tab_hbm.at[sidx_v], rows_v, sem).wait()
            pltpu.sync_copy(rows_v, acc_sh.at[didx_v], add=True)
            return carry

        lax.fori_loop(0, NCH, body, 0)
        plsc.subcore_barrier()

        for cc in range(RPT // RB):
            r0 = s * RPT + cc * RB
            pltpu.sync_copy(acc_sh.at[pl.ds(r0, RB), :], zbuf_v)
            pltpu.sync_copy(zbuf_v, out_hbm.at[c, pl.ds(r0, RB), :])

    return sc_agg


_sc_agg_wide = _make_sc_agg(WPAD)
_sc_agg_feat = _make_sc_agg(D)


def kernel(x, edge_index, W1, b1, W2, b2):
    src = edge_index[0]
    dst = edge_index[1]
    tab1 = _tc1(x, W1, b1)
    agg1 = _sc_agg_wide(tab1, src, dst)
    tab2, deg = _tc2(agg1[0], agg1[1], W2, b2)
    agg2 = _sc_agg_feat(tab2, src, dst)
    x0, xr = _tc3(agg2[0], agg2[1], deg)
    return jnp.concatenate([x0, xr], axis=1)


# speedup vs baseline: 8.2224x; 1.6250x over previous
import functools

import jax
import jax.numpy as jnp
from jax import lax
from jax.experimental import pallas as pl
from jax.experimental.pallas import tpu as pltpu
from jax.experimental.pallas import tpu_sc as plsc

N = 10000
E = 320000
D = 128
WPAD = 136

NC = 2
NS = 16
NW = NC * NS
EPT = E // NW
K = 80
NCH = EPT // K
RPT = N // NS
RB = 25

RBLK = 1000


def _roundtrip_scale(u):
    t2 = jnp.sum(u * u, axis=1, keepdims=True)
    t_raw = jnp.sqrt(t2)
    t = jnp.maximum(t_raw, 1e-7)
    e = jnp.exp(t)
    ei = 1.0 / e
    ch = 0.5 * (e + ei)
    sh = 0.5 * (e - ei)
    x0c = jnp.maximum(ch, 1.0 + 1e-7)
    dist = jnp.log(x0c + jnp.sqrt(x0c * x0c - 1.0))
    nrm = jnp.maximum(sh * (t_raw / t), 1e-7)
    return u * (dist * sh / (t * nrm))


def _tc1_body(x_ref, w_ref, b_ref, o_ref):
    u = _roundtrip_scale(x_ref[...])
    v = jnp.dot(u, w_ref[...], preferred_element_type=jnp.float32) + b_ref[...]
    r = v.shape[0]
    o_ref[...] = jnp.concatenate(
        [v, jnp.ones((r, 1), jnp.float32), jnp.zeros((r, WPAD - D - 1), jnp.float32)],
        axis=1,
    )


def _tc1(x, w, b):
    return pl.pallas_call(
        _tc1_body,
        grid=(N // RBLK,),
        in_specs=[
            pl.BlockSpec((RBLK, D), lambda i: (i, 0)),
            pl.BlockSpec((D, D), lambda i: (0, 0)),
            pl.BlockSpec((D,), lambda i: (0,)),
        ],
        out_specs=pl.BlockSpec((RBLK, WPAD), lambda i: (i, 0)),
        out_shape=jax.ShapeDtypeStruct((N, WPAD), jnp.float32),
    )(x, w, b)


def _tc2_body(p0_ref, p1_ref, w_ref, b_ref, tab_ref, deg_ref):
    s = p0_ref[...] + p1_ref[...]
    agg = s[:, :D]
    deg = jnp.maximum(s[:, D:D + 1], 1.0)
    u = jnp.maximum(agg / deg, 0.0)
    v = jnp.dot(_roundtrip_scale(u), w_ref[...], preferred_element_type=jnp.float32)
    tab_ref[...] = v + b_ref[...]
    deg_ref[...] = deg


def _tc2(p0, p1, w, b):
    return pl.pallas_call(
        _tc2_body,
        grid=(N // RBLK,),
        in_specs=[
            pl.BlockSpec((RBLK, WPAD), lambda i: (i, 0)),
            pl.BlockSpec((RBLK, WPAD), lambda i: (i, 0)),
            pl.BlockSpec((D, D), lambda i: (0, 0)),
            pl.BlockSpec((D,), lambda i: (0,)),
        ],
        out_specs=[
            pl.BlockSpec((RBLK, D), lambda i: (i, 0)),
            pl.BlockSpec((RBLK, 1), lambda i: (i, 0)),
        ],
        out_shape=[
            jax.ShapeDtypeStruct((N, D), jnp.float32),
            jax.ShapeDtypeStruct((N, 1), jnp.float32),
        ],
    )(p0, p1, w, b)


def _tc3_body(p0_ref, p1_ref, deg_ref, x0_ref, xr_ref):
    u = jnp.maximum((p0_ref[...] + p1_ref[...]) / deg_ref[...], 0.0)
    t_raw = jnp.sqrt(jnp.sum(u * u, axis=1, keepdims=True))
    t = jnp.maximum(t_raw, 1e-7)
    e = jnp.exp(t)
    ei = 1.0 / e
    x0_ref[...] = 0.5 * (e + ei)
    xr_ref[...] = 0.5 * (e - ei) * u / t


def _tc3(p0, p1, deg):
    return pl.pallas_call(
        _tc3_body,
        grid=(N // RBLK,),
        in_specs=[
            pl.BlockSpec((RBLK, D), lambda i: (i, 0)),
            pl.BlockSpec((RBLK, D), lambda i: (i, 0)),
            pl.BlockSpec((RBLK, 1), lambda i: (i, 0)),
        ],
        out_specs=[
            pl.BlockSpec((RBLK, 1), lambda i: (i, 0)),
            pl.BlockSpec((RBLK, D), lambda i: (i, 0)),
        ],
        out_shape=[
            jax.ShapeDtypeStruct((N, 1), jnp.float32),
            jax.ShapeDtypeStruct((N, D), jnp.float32),
        ],
    )(p0, p1, deg)


def _make_sc_agg(wd):
    mesh = plsc.VectorSubcoreMesh(core_axis_name="c", subcore_axis_name="s")

    @functools.partial(
        pl.kernel,
        out_type=jax.ShapeDtypeStruct((NC, N, wd), jnp.float32),
        mesh=mesh,
        scratch_types=[
            pltpu.VMEM((NCH, K), jnp.int32),
            pltpu.VMEM((NCH, K), jnp.int32),
            pltpu.VMEM((K, wd), jnp.float32),
            pltpu.VMEM((K, wd), jnp.float32),
            pltpu.VMEM((RB, wd), jnp.float32),
            pltpu.VMEM_SHARED((N, wd), jnp.float32),
            pltpu.SemaphoreType.DMA,
            pltpu.SemaphoreType.DMA,
        ],
        compiler_params=pltpu.CompilerParams(use_tc_tiling_on_sc=False),
    )
    def sc_agg(tab_hbm, src_hbm, dst_hbm, out_hbm,
               sidx_v, didx_v, rows0_v, rows1_v, zbuf_v, acc_sh, sem0, sem1):
        c = lax.axis_index("c")
        s = lax.axis_index("s")
        wid = s * NC + c

        def zrow(r, carry):
            for j in range(wd // 16):
                zbuf_v[r, pl.ds(j * 16, 16)] = jnp.zeros((16,), jnp.float32)
            return carry

        lax.fori_loop(0, RB, zrow, 0)
        for cc in range(RPT // RB):
            pltpu.sync_copy(zbuf_v, acc_sh.at[pl.ds(s * RPT + cc * RB, RB), :])

        pltpu.sync_copy(src_hbm.at[wid], sidx_v)
        pltpu.sync_copy(dst_hbm.at[wid], didx_v)
        plsc.subcore_barrier()

        pltpu.async_copy(tab_hbm.at[sidx_v.at[0]], rows0_v, sem0)

        def pair(i, carry):
            j0 = 2 * i
            pltpu.make_async_copy(tab_hbm.at[sidx_v.at[j0]], rows0_v, sem0).wait()
            pltpu.async_copy(tab_hbm.at[sidx_v.at[j0 + 1]], rows1_v, sem1)
            pltpu.sync_copy(rows0_v, acc_sh.at[didx_v.at[j0]], add=True)
            pltpu.make_async_copy(tab_hbm.at[sidx_v.at[j0 + 1]], rows1_v, sem1).wait()
            pltpu.async_copy(tab_hbm.at[sidx_v.at[j0 + 2]], rows0_v, sem0)
            pltpu.sync_copy(rows1_v, acc_sh.at[didx_v.at[j0 + 1]], add=True)
            return carry

        lax.fori_loop(0, (NCH - 1) // 2, pair, 0)
        pltpu.make_async_copy(tab_hbm.at[sidx_v.at[NCH - 1]], rows0_v, sem0).wait()
        pltpu.sync_copy(rows0_v, acc_sh.at[didx_v.at[NCH - 1]], add=True)
        plsc.subcore_barrier()

        for cc in range(RPT // RB):
            r0 = s * RPT + cc * RB
            pltpu.sync_copy(acc_sh.at[pl.ds(r0, RB), :], zbuf_v)
            pltpu.sync_copy(zbuf_v, out_hbm.at[c, pl.ds(r0, RB), :])

    return sc_agg


_sc_agg_wide = _make_sc_agg(WPAD)
_sc_agg_feat = _make_sc_agg(D)


def kernel(x, edge_index, W1, b1, W2, b2):
    src = edge_index[0].reshape(NW, NCH, K)
    dst = edge_index[1].reshape(NW, NCH, K)
    tab1 = _tc1(x, W1, b1)
    agg1 = _sc_agg_wide(tab1, src, dst)
    tab2, deg = _tc2(agg1[0], agg1[1], W2, b2)
    agg2 = _sc_agg_feat(tab2, src, dst)
    x0, xr = _tc3(agg2[0], agg2[1], deg)
    return jnp.concatenate([x0, xr], axis=1)
